# Initial kernel scaffold; baseline (speedup 1.0000x reference)
import jax
import jax.numpy as jnp
from jax.experimental import pallas as pl


def kernel(hidden_states, gate_w, w_gate_proj, w_up_proj, w_down_proj):
    raise NotImplementedError("write your pallas kernel here")


# speedup vs baseline: 1.1546x; 1.1546x over previous
import jax
import jax.numpy as jnp
from jax.experimental import pallas as pl
from jax.experimental.pallas import tpu as pltpu

_NUM_EXPERTS = 64
_TOP_K = 8


def _moe_body(x_ref, gw_ref, wg_ref, wu_ref, wd_ref, out_ref, coef_ref):
    e = pl.program_id(0)
    T = x_ref.shape[0]
    E = _NUM_EXPERTS

    @pl.when(e == 0)
    def _router():
        x = x_ref[...]
        logits = jax.lax.dot_general(
            x, gw_ref[...], (((1,), (1,)), ((), ())),
            preferred_element_type=jnp.float32,
        )
        col = jax.lax.broadcasted_iota(jnp.int32, (T, E), 1)
        neg = jnp.float32(-1e30)
        work = logits
        mask = jnp.zeros((T, E), dtype=jnp.bool_)
        for _ in range(_TOP_K):
            m = jnp.max(work, axis=1, keepdims=True)
            is_max = work == m
            j = jnp.min(jnp.where(is_max, col, E), axis=1, keepdims=True)
            pick = col == j
            mask = mask | pick
            work = jnp.where(pick, neg, work)
        sel = jnp.where(mask, logits, neg)
        mx = jnp.max(sel, axis=1, keepdims=True)
        ex = jnp.where(mask, jnp.exp(logits - mx), 0.0)
        coef_ref[...] = ex / jnp.sum(ex, axis=1, keepdims=True)

    x = x_ref[...]
    g = jax.lax.dot_general(
        x, wg_ref[0], (((1,), (1,)), ((), ())),
        preferred_element_type=jnp.float32,
    )
    u = jax.lax.dot_general(
        x, wu_ref[0], (((1,), (1,)), ((), ())),
        preferred_element_type=jnp.float32,
    )
    h = g * jax.lax.logistic(g) * u
    lane = jax.lax.broadcasted_iota(jnp.int32, (T, E), 1)
    coef_col = jnp.sum(
        jnp.where(lane == e, coef_ref[...], 0.0), axis=1, keepdims=True
    )
    hs = h * coef_col
    y = jax.lax.dot_general(
        hs, wd_ref[0], (((1,), (1,)), ((), ())),
        preferred_element_type=jnp.float32,
    )

    @pl.when(e == 0)
    def _init():
        out_ref[...] = y

    @pl.when(e != 0)
    def _acc():
        out_ref[...] += y


def kernel(hidden_states, gate_w, w_gate_proj, w_up_proj, w_down_proj):
    B, S, D = hidden_states.shape
    T = B * S
    E, F, _ = w_gate_proj.shape
    x = hidden_states.reshape(T, D)

    out = pl.pallas_call(
        _moe_body,
        grid=(E,),
        in_specs=[
            pl.BlockSpec((T, D), lambda e: (0, 0)),
            pl.BlockSpec((E, D), lambda e: (0, 0)),
            pl.BlockSpec((1, F, D), lambda e: (e, 0, 0)),
            pl.BlockSpec((1, F, D), lambda e: (e, 0, 0)),
            pl.BlockSpec((1, D, F), lambda e: (e, 0, 0)),
        ],
        out_specs=pl.BlockSpec((T, D), lambda e: (0, 0)),
        out_shape=jax.ShapeDtypeStruct((T, D), jnp.float32),
        scratch_shapes=[pltpu.VMEM((T, E), jnp.float32)],
    )(x, gate_w, w_gate_proj, w_up_proj, w_down_proj)
    return out.reshape(B, S, D)
